# Initial kernel scaffold; baseline (speedup 1.0000x reference)
import jax
import jax.numpy as jnp
from jax.experimental import pallas as pl


def kernel(x, edge_index, batch, W_rel0, b_rel0, W_rel1, b_rel1, W_rel2, b_rel2, W_root0, W_root1, W_root2, W_out, b_out):
    raise NotImplementedError("write your pallas kernel here")


# speedup vs baseline: 7.9956x; 7.9956x over previous
import functools

import jax
import jax.numpy as jnp
from jax import lax
from jax.experimental import pallas as pl
from jax.experimental.pallas import tpu as pltpu
from jax.experimental.pallas import tpu_sc as plsc

N_NODES = 10000
N_EDGES = 320000
D = 128
N_GRAPHS = 64
OUT_DIM = 24

NC = 2
NS = 16
NW = NC * NS
E_PER_W = N_EDGES // NW
CHUNK = 128
PAD_PER_W = -E_PER_W % CHUNK
EPW_PAD = E_PER_W + PAD_PER_W
N_CHUNKS = EPW_PAD // CHUNK
N_DUMMY = 240
ACC_ROWS = N_NODES + N_DUMMY
ROWS_PER_TILE = ACC_ROWS // NS
_MESH = plsc.VectorSubcoreMesh(core_axis_name="c", subcore_axis_name="s",
                               num_cores=NC, num_subcores=NS)


def _agg_body(h_hbm, src_hbm, dst_hbm, z_hbm, out_hbm,
              acc, srcv, dstv, stag, sem):
    c = lax.axis_index("c")
    s = lax.axis_index("s")
    wid = s * NC + c

    pltpu.sync_copy(z_hbm, acc.at[pl.ds(s * ROWS_PER_TILE, ROWS_PER_TILE)])
    pltpu.sync_copy(src_hbm.at[wid], srcv)
    pltpu.sync_copy(dst_hbm.at[wid], dstv)
    plsc.subcore_barrier()

    def step(i, carry):
        pltpu.async_copy(h_hbm.at[srcv.at[i]], stag, sem).wait()
        pltpu.sync_copy(stag, acc.at[dstv.at[i]], add=True)
        return carry

    lax.fori_loop(0, N_CHUNKS, step, 0)
    plsc.subcore_barrier()

    pltpu.sync_copy(
        acc.at[pl.ds(s * ROWS_PER_TILE, ROWS_PER_TILE)],
        out_hbm.at[pl.ds(c * ACC_ROWS + s * ROWS_PER_TILE, ROWS_PER_TILE)])


def _counts_body(dst_hbm, z_hbm, cnt_hbm, cacc, dstv, stag):
    c = lax.axis_index("c")
    s = lax.axis_index("s")
    wid = s * NC + c

    pltpu.sync_copy(z_hbm, cacc.at[pl.ds(s * ROWS_PER_TILE, ROWS_PER_TILE)])
    pltpu.sync_copy(dst_hbm.at[wid], dstv)
    one16 = jnp.ones((16,), jnp.float32)

    def fill(r, carry):
        for g in range(D // 16):
            stag[r, pl.ds(g * 16, 16)] = one16
        return carry

    lax.fori_loop(0, CHUNK, fill, 0)
    plsc.subcore_barrier()

    def step(i, carry):
        pltpu.sync_copy(stag, cacc.at[dstv.at[i]], add=True)
        return carry

    lax.fori_loop(0, N_CHUNKS, step, 0)
    plsc.subcore_barrier()
    pltpu.sync_copy(
        cacc.at[pl.ds(s * ROWS_PER_TILE, ROWS_PER_TILE)],
        cnt_hbm.at[pl.ds(c * ACC_ROWS + s * ROWS_PER_TILE, ROWS_PER_TILE)])


_agg = pl.kernel(
    _agg_body,
    out_type=jax.ShapeDtypeStruct((NC * ACC_ROWS, D), jnp.float32),
    mesh=_MESH,
    scratch_types=[
        pltpu.VMEM_SHARED((ACC_ROWS, D), jnp.float32),
        pltpu.VMEM((N_CHUNKS, CHUNK), jnp.int32),
        pltpu.VMEM((N_CHUNKS, CHUNK), jnp.int32),
        pltpu.VMEM((CHUNK, D), jnp.float32),
        pltpu.SemaphoreType.DMA,
    ],
)

_counts = pl.kernel(
    _counts_body,
    out_type=jax.ShapeDtypeStruct((NC * ACC_ROWS, D), jnp.float32),
    mesh=_MESH,
    scratch_types=[
        pltpu.VMEM_SHARED((ACC_ROWS, D), jnp.float32),
        pltpu.VMEM((N_CHUNKS, CHUNK), jnp.int32),
        pltpu.VMEM((CHUNK, D), jnp.float32),
    ],
)


def _combine_body(agg_ref, cnt_ref, h_ref, wrel_ref, brel_ref, wroot_ref, o_ref):
    summed = agg_ref[0:N_NODES, :] + agg_ref[ACC_ROWS:ACC_ROWS + N_NODES, :]
    cnt = cnt_ref[0:N_NODES, 0:1] + cnt_ref[ACC_ROWS:ACC_ROWS + N_NODES, 0:1]
    mean = summed * (1.0 / jnp.maximum(cnt, 1.0))
    z = lax.dot_general(mean, wrel_ref[...], (((1,), (1,)), ((), ())),
                        preferred_element_type=jnp.float32)
    z = z + brel_ref[...]
    z = z + lax.dot_general(h_ref[...], wroot_ref[...], (((1,), (1,)), ((), ())),
                            preferred_element_type=jnp.float32)
    o_ref[...] = jnp.maximum(z, 0.0)


_combine = pl.pallas_call(
    _combine_body,
    out_shape=jax.ShapeDtypeStruct((N_NODES, D), jnp.float32),
)


def _final_body(agg_ref, cnt_ref, h_ref, wrel_ref, brel_ref, wroot_ref,
                batch_ref, wout_ref, bout_ref, o_ref):
    summed = agg_ref[0:N_NODES, :] + agg_ref[ACC_ROWS:ACC_ROWS + N_NODES, :]
    cnt = cnt_ref[0:N_NODES, 0:1] + cnt_ref[ACC_ROWS:ACC_ROWS + N_NODES, 0:1]
    mean = summed * (1.0 / jnp.maximum(cnt, 1.0))
    z = lax.dot_general(mean, wrel_ref[...], (((1,), (1,)), ((), ())),
                        preferred_element_type=jnp.float32)
    z = z + brel_ref[...]
    z = z + lax.dot_general(h_ref[...], wroot_ref[...], (((1,), (1,)), ((), ())),
                            preferred_element_type=jnp.float32)
    h3 = jnp.maximum(z, 0.0)
    gids = lax.broadcasted_iota(jnp.int32, (N_NODES, N_GRAPHS), 1)
    mask = (batch_ref[...] == gids).astype(jnp.float32)
    gsum = lax.dot_general(mask, h3, (((0,), (0,)), ((), ())),
                           preferred_element_type=jnp.float32)
    gcnt = jnp.sum(mask, axis=0)[:, None]
    g = gsum * (1.0 / jnp.maximum(gcnt, 1.0))
    o_ref[...] = lax.dot_general(g, wout_ref[...], (((1,), (1,)), ((), ())),
                                 preferred_element_type=jnp.float32) + bout_ref[...]


_final = pl.pallas_call(
    _final_body,
    out_shape=jax.ShapeDtypeStruct((N_GRAPHS, OUT_DIM), jnp.float32),
)


def kernel(x, edge_index, batch, W_rel0, b_rel0, W_rel1, b_rel1, W_rel2, b_rel2,
           W_root0, W_root1, W_root2, W_out, b_out):
    src = edge_index[0].astype(jnp.int32).reshape(NW, E_PER_W)
    dst = edge_index[1].astype(jnp.int32).reshape(NW, E_PER_W)
    pad_src = jnp.broadcast_to(
        (jnp.arange(PAD_PER_W, dtype=jnp.int32) * 41) % N_NODES, (NW, PAD_PER_W))
    pad_dst = jnp.broadcast_to(
        N_NODES + (jnp.arange(PAD_PER_W, dtype=jnp.int32) % N_DUMMY),
        (NW, PAD_PER_W))
    src3 = jnp.concatenate([src, pad_src], axis=1).reshape(NW, N_CHUNKS, CHUNK)
    dst3 = jnp.concatenate([dst, pad_dst], axis=1).reshape(NW, N_CHUNKS, CHUNK)

    z = jnp.zeros((ROWS_PER_TILE, D), jnp.float32)

    cnt = _counts(dst3, z)
    agg0 = _agg(x, src3, dst3, z)
    h1 = _combine(agg0, cnt, x, W_rel0, b_rel0.reshape(1, D), W_root0)
    agg1 = _agg(h1, src3, dst3, z)
    h2 = _combine(agg1, cnt, h1, W_rel1, b_rel1.reshape(1, D), W_root1)
    agg2 = _agg(h2, src3, dst3, z)
    return _final(agg2, cnt, h2, W_rel2, b_rel2.reshape(1, D), W_root2,
                  batch.astype(jnp.int32).reshape(N_NODES, 1), W_out,
                  b_out.reshape(1, OUT_DIM))


# speedup vs baseline: 10.3466x; 1.2940x over previous
import functools

import jax
import jax.numpy as jnp
from jax import lax
from jax.experimental import pallas as pl
from jax.experimental.pallas import tpu as pltpu
from jax.experimental.pallas import tpu_sc as plsc

N_NODES = 10000
N_EDGES = 320000
D = 128
N_GRAPHS = 64
OUT_DIM = 24

NC = 2
NS = 16
NW = NC * NS
E_PER_W = N_EDGES // NW
CHUNK = 128
PAD_PER_W = -E_PER_W % CHUNK
EPW_PAD = E_PER_W + PAD_PER_W
N_CHUNKS = EPW_PAD // CHUNK
N_DUMMY = 240
ACC_ROWS = N_NODES + N_DUMMY
ROWS_PER_TILE = ACC_ROWS // NS
_MESH = plsc.VectorSubcoreMesh(core_axis_name="c", subcore_axis_name="s",
                               num_cores=NC, num_subcores=NS)


def _agg_body(h_hbm, ei_hbm, z_hbm, out_hbm,
              acc, idx0, idx1, stag0, stag1, sem_i0, sem_i1, sem_g0, sem_g1):
    c = lax.axis_index("c")
    s = lax.axis_index("s")
    wid = s * NC + c

    pltpu.sync_copy(z_hbm, acc.at[pl.ds(s * ROWS_PER_TILE, ROWS_PER_TILE)])

    def start_i(i, buf, sem):
        pltpu.async_copy(ei_hbm.at[wid, i], buf, sem)

    def wait_i(buf, sem):
        pltpu.make_async_copy(ei_hbm.at[0, 0], buf, sem).wait()

    def start_g(buf_idx, buf, sem):
        pltpu.async_copy(h_hbm.at[buf_idx.at[0]], buf, sem)

    def wait_g(buf, sem):
        pltpu.make_async_copy(h_hbm.at[pl.ds(0, CHUNK)], buf, sem).wait()

    start_i(0, idx0, sem_i0)
    wait_i(idx0, sem_i0)
    start_g(idx0, stag0, sem_g0)
    start_i(1, idx1, sem_i1)
    plsc.subcore_barrier()

    def step(p, carry):
        a = 2 * p
        wait_i(idx1, sem_i1)
        wait_g(stag0, sem_g0)
        start_g(idx1, stag1, sem_g1)
        pltpu.sync_copy(stag0, acc.at[idx0.at[1]], add=True)

        @pl.when(a + 2 < N_CHUNKS)
        def _():
            start_i(a + 2, idx0, sem_i0)

        wait_g(stag1, sem_g1)

        @pl.when(a + 2 < N_CHUNKS)
        def _():
            wait_i(idx0, sem_i0)
            start_g(idx0, stag0, sem_g0)

        pltpu.sync_copy(stag1, acc.at[idx1.at[1]], add=True)

        @pl.when(a + 3 < N_CHUNKS)
        def _():
            start_i(a + 3, idx1, sem_i1)

        return carry

    lax.fori_loop(0, N_CHUNKS // 2, step, 0)
    plsc.subcore_barrier()

    pltpu.sync_copy(
        acc.at[pl.ds(s * ROWS_PER_TILE, ROWS_PER_TILE)],
        out_hbm.at[pl.ds(c * ACC_ROWS + s * ROWS_PER_TILE, ROWS_PER_TILE)])


def _counts_body(dst_hbm, z_hbm, cnt_hbm, cacc, dstv, stag):
    c = lax.axis_index("c")
    s = lax.axis_index("s")
    wid = s * NC + c

    pltpu.sync_copy(z_hbm, cacc.at[pl.ds(s * ROWS_PER_TILE, ROWS_PER_TILE)])
    pltpu.sync_copy(dst_hbm.at[wid], dstv)
    one16 = jnp.ones((16,), jnp.float32)

    def fill(r, carry):
        for g in range(D // 16):
            stag[r, pl.ds(g * 16, 16)] = one16
        return carry

    lax.fori_loop(0, CHUNK, fill, 0)
    plsc.subcore_barrier()

    def step(i, carry):
        pltpu.sync_copy(stag, cacc.at[dstv.at[i]], add=True)
        return carry

    lax.fori_loop(0, N_CHUNKS, step, 0)
    plsc.subcore_barrier()
    pltpu.sync_copy(
        cacc.at[pl.ds(s * ROWS_PER_TILE, ROWS_PER_TILE)],
        cnt_hbm.at[pl.ds(c * ACC_ROWS + s * ROWS_PER_TILE, ROWS_PER_TILE)])


_agg = pl.kernel(
    _agg_body,
    out_type=jax.ShapeDtypeStruct((NC * ACC_ROWS, D), jnp.float32),
    mesh=_MESH,
    scratch_types=[
        pltpu.VMEM_SHARED((ACC_ROWS, D), jnp.float32),
        pltpu.VMEM((2, CHUNK), jnp.int32),
        pltpu.VMEM((2, CHUNK), jnp.int32),
        pltpu.VMEM((CHUNK, D), jnp.float32),
        pltpu.VMEM((CHUNK, D), jnp.float32),
        pltpu.SemaphoreType.DMA,
        pltpu.SemaphoreType.DMA,
        pltpu.SemaphoreType.DMA,
        pltpu.SemaphoreType.DMA,
    ],
)

_counts = pl.kernel(
    _counts_body,
    out_type=jax.ShapeDtypeStruct((NC * ACC_ROWS, D), jnp.float32),
    mesh=_MESH,
    scratch_types=[
        pltpu.VMEM_SHARED((ACC_ROWS, D), jnp.float32),
        pltpu.VMEM((N_CHUNKS, CHUNK), jnp.int32),
        pltpu.VMEM((CHUNK, D), jnp.float32),
    ],
)


def _combine_body(agg_ref, cnt_ref, h_ref, wrel_ref, brel_ref, wroot_ref, o_ref):
    summed = agg_ref[0:N_NODES, :] + agg_ref[ACC_ROWS:ACC_ROWS + N_NODES, :]
    cnt = cnt_ref[0:N_NODES, 0:1] + cnt_ref[ACC_ROWS:ACC_ROWS + N_NODES, 0:1]
    mean = summed * (1.0 / jnp.maximum(cnt, 1.0))
    z = lax.dot_general(mean, wrel_ref[...], (((1,), (1,)), ((), ())),
                        preferred_element_type=jnp.float32)
    z = z + brel_ref[...]
    z = z + lax.dot_general(h_ref[...], wroot_ref[...], (((1,), (1,)), ((), ())),
                            preferred_element_type=jnp.float32)
    o_ref[...] = jnp.maximum(z, 0.0)


_combine = pl.pallas_call(
    _combine_body,
    out_shape=jax.ShapeDtypeStruct((N_NODES, D), jnp.float32),
)


def _final_body(agg_ref, cnt_ref, h_ref, wrel_ref, brel_ref, wroot_ref,
                batch_ref, wout_ref, bout_ref, o_ref):
    summed = agg_ref[0:N_NODES, :] + agg_ref[ACC_ROWS:ACC_ROWS + N_NODES, :]
    cnt = cnt_ref[0:N_NODES, 0:1] + cnt_ref[ACC_ROWS:ACC_ROWS + N_NODES, 0:1]
    mean = summed * (1.0 / jnp.maximum(cnt, 1.0))
    z = lax.dot_general(mean, wrel_ref[...], (((1,), (1,)), ((), ())),
                        preferred_element_type=jnp.float32)
    z = z + brel_ref[...]
    z = z + lax.dot_general(h_ref[...], wroot_ref[...], (((1,), (1,)), ((), ())),
                            preferred_element_type=jnp.float32)
    h3 = jnp.maximum(z, 0.0)
    gids = lax.broadcasted_iota(jnp.int32, (N_NODES, N_GRAPHS), 1)
    mask = (batch_ref[...] == gids).astype(jnp.float32)
    gsum = lax.dot_general(mask, h3, (((0,), (0,)), ((), ())),
                           preferred_element_type=jnp.float32)
    gcnt = jnp.sum(mask, axis=0)[:, None]
    g = gsum * (1.0 / jnp.maximum(gcnt, 1.0))
    o_ref[...] = lax.dot_general(g, wout_ref[...], (((1,), (1,)), ((), ())),
                                 preferred_element_type=jnp.float32) + bout_ref[...]


_final = pl.pallas_call(
    _final_body,
    out_shape=jax.ShapeDtypeStruct((N_GRAPHS, OUT_DIM), jnp.float32),
)


def kernel(x, edge_index, batch, W_rel0, b_rel0, W_rel1, b_rel1, W_rel2, b_rel2,
           W_root0, W_root1, W_root2, W_out, b_out):
    src = edge_index[0].astype(jnp.int32).reshape(NW, E_PER_W)
    dst = edge_index[1].astype(jnp.int32).reshape(NW, E_PER_W)
    pad_src = jnp.broadcast_to(
        (jnp.arange(PAD_PER_W, dtype=jnp.int32) * 41) % N_NODES, (NW, PAD_PER_W))
    pad_dst = jnp.broadcast_to(
        N_NODES + (jnp.arange(PAD_PER_W, dtype=jnp.int32) % N_DUMMY),
        (NW, PAD_PER_W))
    src3 = jnp.concatenate([src, pad_src], axis=1).reshape(NW, N_CHUNKS, CHUNK)
    dst3 = jnp.concatenate([dst, pad_dst], axis=1).reshape(NW, N_CHUNKS, CHUNK)
    ei3 = jnp.stack([src3, dst3], axis=2)

    z = jnp.zeros((ROWS_PER_TILE, D), jnp.float32)

    cnt = _counts(dst3, z)
    agg0 = _agg(x, ei3, z)
    h1 = _combine(agg0, cnt, x, W_rel0, b_rel0.reshape(1, D), W_root0)
    agg1 = _agg(h1, ei3, z)
    h2 = _combine(agg1, cnt, h1, W_rel1, b_rel1.reshape(1, D), W_root1)
    agg2 = _agg(h2, ei3, z)
    return _final(agg2, cnt, h2, W_rel2, b_rel2.reshape(1, D), W_root2,
                  batch.astype(jnp.int32).reshape(N_NODES, 1), W_out,
                  b_out.reshape(1, OUT_DIM))
